# baseline (speedup 1.0000x reference)
import functools

import jax
import jax.numpy as jnp
from jax import lax
from jax.experimental import pallas as pl
from jax.experimental.pallas import tpu as pltpu
from jax.experimental.pallas import tpu_sc as plsc

VOCAB = 1000000
EMB = 64
PAD_EMB = 128
BATCH = 4096
SEQ = 200
NTOK = BATCH * SEQ

NC = 2
NS = 16
NW = NC * NS
PER_W = NTOK // NW

GDMA = 128
CHUNK = 256
N_GATH = CHUNK // GDMA
N_CHUNKS = PER_W // CHUNK
LANES = 16


@functools.partial(
    pl.kernel,
    mesh=plsc.VectorSubcoreMesh(core_axis_name="c", subcore_axis_name="s"),
    out_type=jax.ShapeDtypeStruct((NTOK, EMB), jnp.float32),
    scratch_types=[
        pltpu.VMEM((2, CHUNK), jnp.int32),
        pltpu.VMEM((2, CHUNK, PAD_EMB), jnp.float32),
        pltpu.VMEM((CHUNK, EMB), jnp.float32),
        pltpu.SemaphoreType.DMA,
        pltpu.SemaphoreType.DMA,
    ],
)
def _emb_lookup(idx_hbm, table_hbm, out_hbm, idx_v, rows_v, rows64_v,
                sem0, sem1):
    wid = lax.axis_index("s") * NC + lax.axis_index("c")
    tok_base = wid * PER_W
    sems = (sem0, sem1)

    def stage_and_fire(g, b):
        pltpu.sync_copy(idx_hbm.at[pl.ds(tok_base + g * CHUNK, CHUNK)],
                        idx_v.at[b])
        for j in range(N_GATH):
            pltpu.async_copy(table_hbm.at[idx_v.at[b].at[pl.ds(j * GDMA, GDMA)]],
                             rows_v.at[b].at[pl.ds(j * GDMA, GDMA)], sems[b])

    def drain_gathers(b):
        for j in range(N_GATH):
            pltpu.make_async_copy(
                table_hbm.at[idx_v.at[b].at[pl.ds(j * GDMA, GDMA)]],
                rows_v.at[b].at[pl.ds(j * GDMA, GDMA)], sems[b]).wait()

    def compact_and_store(g, b):
        def row_body(t, carry):
            for k in range(EMB // LANES):
                rows64_v[t, pl.ds(k * LANES, LANES)] = (
                    rows_v.at[b][t, pl.ds(k * LANES, LANES)])
            return carry

        lax.fori_loop(0, CHUNK, row_body, 0)
        pltpu.sync_copy(rows64_v,
                        out_hbm.at[pl.ds(tok_base + g * CHUNK, CHUNK)])

    stage_and_fire(0, 0)
    stage_and_fire(1, 1)

    def body(p, carry):
        for b in range(2):
            g = 2 * p + b
            drain_gathers(b)
            compact_and_store(g, b)
            stage_and_fire(g + 2, b)
        return carry

    lax.fori_loop(0, N_CHUNKS // 2 - 1, body, 0)

    for b in range(2):
        g = N_CHUNKS - 2 + b
        drain_gathers(b)
        compact_and_store(g, b)


_PAD_BLK = 8000


def _pad_body(t_ref, o_ref):
    o_ref[...] = jnp.pad(t_ref[...], ((0, 0), (0, PAD_EMB - EMB)))


_pad_table = pl.pallas_call(
    _pad_body,
    grid=(VOCAB // _PAD_BLK,),
    in_specs=[pl.BlockSpec((_PAD_BLK, EMB), lambda i: (i, 0))],
    out_specs=pl.BlockSpec((_PAD_BLK, PAD_EMB), lambda i: (i, 0)),
    out_shape=jax.ShapeDtypeStruct((VOCAB, PAD_EMB), jnp.float32),
)


def kernel(x, table):
    idx = x.reshape(NTOK).astype(jnp.int32)
    table_padded = _pad_table(table)
    out = _emb_lookup(idx, table_padded)
    return out.reshape(BATCH, SEQ, EMB)


# speedup vs baseline: 1.0187x; 1.0187x over previous
import functools

import jax
import jax.numpy as jnp
from jax import lax
from jax.experimental import pallas as pl
from jax.experimental.pallas import tpu as pltpu
from jax.experimental.pallas import tpu_sc as plsc

VOCAB = 1000000
EMB = 64
PAD_EMB = 128
BATCH = 4096
SEQ = 200
NTOK = BATCH * SEQ

NC = 2
NS = 16
NW = NC * NS
ROWS_W = BATCH // NW
PER_W = NTOK // NW

GDMA = 128
CHUNK = 256
N_GATH = CHUNK // GDMA
N_CHUNKS = PER_W // CHUNK
LANES = 16
HALF = CHUNK // 2


@functools.partial(
    pl.kernel,
    mesh=plsc.VectorSubcoreMesh(core_axis_name="c", subcore_axis_name="s"),
    compiler_params=pltpu.CompilerParams(needs_layout_passes=False),
    out_type=jax.ShapeDtypeStruct((NTOK, EMB), jnp.float32),
    scratch_types=[
        pltpu.VMEM((ROWS_W, SEQ), jnp.int32),
        pltpu.VMEM((2, CHUNK), jnp.int32),
        pltpu.VMEM((2, CHUNK, PAD_EMB), jnp.float32),
        pltpu.VMEM((HALF, EMB), jnp.float32),
        pltpu.SemaphoreType.DMA,
        pltpu.SemaphoreType.DMA,
    ],
)
def _emb_lookup(x_hbm, table_hbm, out_hbm, x_v, idx_v, rows_v, rows64_v,
                sem0, sem1):
    wid = lax.axis_index("s") * NC + lax.axis_index("c")
    tok_base = wid * PER_W
    sems = (sem0, sem1)

    pltpu.sync_copy(x_hbm.at[pl.ds(wid * ROWS_W, ROWS_W)], x_v)

    def stage_and_fire(g, b):
        for k in range(CHUNK // LANES):
            u = g * CHUNK + k * LANES + lax.iota(jnp.int32, LANES)
            ids = plsc.load_gather(x_v, [lax.div(u, SEQ), lax.rem(u, SEQ)])
            idx_v[b, pl.ds(k * LANES, LANES)] = ids
        for j in range(N_GATH):
            pltpu.async_copy(table_hbm.at[idx_v.at[b].at[pl.ds(j * GDMA, GDMA)]],
                             rows_v.at[b].at[pl.ds(j * GDMA, GDMA)], sems[b])

    def drain_gathers(b):
        for j in range(N_GATH):
            pltpu.make_async_copy(
                table_hbm.at[idx_v.at[b].at[pl.ds(j * GDMA, GDMA)]],
                rows_v.at[b].at[pl.ds(j * GDMA, GDMA)], sems[b]).wait()

    def compact_and_store(g, b):
        for h in range(2):
            def row_body(t, carry):
                for k in range(EMB // LANES):
                    rows64_v[t, pl.ds(k * LANES, LANES)] = (
                        rows_v.at[b][h * HALF + t, pl.ds(k * LANES, LANES)])
                return carry

            lax.fori_loop(0, HALF, row_body, 0)
            pltpu.sync_copy(
                rows64_v,
                out_hbm.at[pl.ds(tok_base + g * CHUNK + h * HALF, HALF)])

    stage_and_fire(0, 0)
    stage_and_fire(1, 1)

    def body(p, carry):
        for b in range(2):
            g = 2 * p + b
            drain_gathers(b)
            compact_and_store(g, b)
            stage_and_fire(g + 2, b)
        return carry

    lax.fori_loop(0, N_CHUNKS // 2 - 1, body, 0)

    for b in range(2):
        g = N_CHUNKS - 2 + b
        drain_gathers(b)
        compact_and_store(g, b)


_PAD_BLK = 25000


def _pad_body(t_ref, o_ref):
    o_ref[...] = jnp.pad(t_ref[...], ((0, 0), (0, PAD_EMB - EMB)))


_pad_table = pl.pallas_call(
    _pad_body,
    grid=(VOCAB // _PAD_BLK,),
    in_specs=[pl.BlockSpec((_PAD_BLK, EMB), lambda i: (i, 0))],
    out_specs=pl.BlockSpec((_PAD_BLK, PAD_EMB), lambda i: (i, 0)),
    out_shape=jax.ShapeDtypeStruct((VOCAB, PAD_EMB), jnp.float32),
)


def kernel(x, table):
    table_padded = _pad_table(table)
    out = _emb_lookup(x.astype(jnp.int32), table_padded)
    return out.reshape(BATCH, SEQ, EMB)


# speedup vs baseline: 1.1479x; 1.1268x over previous
import functools

import jax
import jax.numpy as jnp
from jax import lax
from jax.experimental import pallas as pl
from jax.experimental.pallas import tpu as pltpu
from jax.experimental.pallas import tpu_sc as plsc

VOCAB = 1000000
EMB = 64
PAD_EMB = 128
BATCH = 4096
SEQ = 200
NTOK = BATCH * SEQ

NC = 2
NS = 16
NW = NC * NS
ROWS_W = BATCH // NW
PER_W = NTOK // NW

GDMA = 128
CHUNK = 256
N_GATH = CHUNK // GDMA
N_CHUNKS = PER_W // CHUNK
LANES = 16
HALF = CHUNK // 2


@functools.partial(
    pl.kernel,
    mesh=plsc.VectorSubcoreMesh(core_axis_name="c", subcore_axis_name="s"),
    compiler_params=pltpu.CompilerParams(needs_layout_passes=False),
    out_type=jax.ShapeDtypeStruct((NTOK, EMB), jnp.float32),
    scratch_types=[
        pltpu.VMEM((ROWS_W, SEQ), jnp.int32),
        pltpu.VMEM((2, CHUNK), jnp.int32),
        pltpu.VMEM((2, CHUNK, PAD_EMB), jnp.float32),
        pltpu.VMEM((HALF, EMB), jnp.float32),
        pltpu.SemaphoreType.DMA,
        pltpu.SemaphoreType.DMA,
    ],
)
def _emb_lookup(x_hbm, table_hbm, out_hbm, x_v, idx_v, rows_v, rows64_v,
                sem0, sem1):
    wid = lax.axis_index("s") * NC + lax.axis_index("c")
    tok_base = wid * PER_W
    sems = (sem0, sem1)

    pltpu.sync_copy(x_hbm.at[pl.ds(wid * ROWS_W, ROWS_W)], x_v)

    def stage_and_fire(g, b):
        for k in range(CHUNK // LANES):
            u = g * CHUNK + k * LANES + lax.iota(jnp.int32, LANES)
            ids = plsc.load_gather(x_v, [lax.div(u, SEQ), lax.rem(u, SEQ)])
            idx_v[b, pl.ds(k * LANES, LANES)] = ids
        for j in range(N_GATH):
            pltpu.async_copy(table_hbm.at[idx_v.at[b].at[pl.ds(j * GDMA, GDMA)]],
                             rows_v.at[b].at[pl.ds(j * GDMA, GDMA)], sems[b])

    def drain_gathers(b):
        for j in range(N_GATH):
            pltpu.make_async_copy(
                table_hbm.at[idx_v.at[b].at[pl.ds(j * GDMA, GDMA)]],
                rows_v.at[b].at[pl.ds(j * GDMA, GDMA)], sems[b]).wait()

    def compact_and_store(g, b):
        for h in range(2):
            def row_body(t, carry):
                for k in range(EMB // LANES):
                    rows64_v[t, pl.ds(k * LANES, LANES)] = (
                        rows_v.at[b][h * HALF + t, pl.ds(k * LANES, LANES)])
                return carry

            lax.fori_loop(0, HALF, row_body, 0)
            pltpu.sync_copy(
                rows64_v,
                out_hbm.at[pl.ds(tok_base + g * CHUNK + h * HALF, HALF)])

    stage_and_fire(0, 0)
    stage_and_fire(1, 1)

    def body(p, carry):
        for b in range(2):
            g = 2 * p + b
            drain_gathers(b)
            compact_and_store(g, b)
            stage_and_fire(g + 2, b)
        return carry

    lax.fori_loop(0, N_CHUNKS // 2 - 1, body, 0)

    for b in range(2):
        g = N_CHUNKS - 2 + b
        drain_gathers(b)
        compact_and_store(g, b)


def kernel(x, table):
    table_padded = jnp.pad(table, ((0, 0), (0, PAD_EMB - EMB)))
    out = _emb_lookup(x.astype(jnp.int32), table_padded)
    return out.reshape(BATCH, SEQ, EMB)
